# baseline (speedup 1.0000x reference)
import functools

import jax
import jax.numpy as jnp
from jax import lax
from jax.experimental import pallas as pl
from jax.experimental.pallas import tpu as pltpu
from jax.experimental.pallas import tpu_sc as plsc

NUM_OBJ_CLS = 151
NUM_REL_CLS = 51
REL_PAD = 128
HIDDEN = 512
N_OBJ = 1000
N_PAD = 1024
N_REL = 4096
NMS_THRESH = 0.5

SC_NC = 2
SC_NS = 16
SC_L = 16
SC_NW = SC_NC * SC_NS
BPW = N_REL // SC_NW


def _tc_body(feats_ref, w_obj_ref, b_row_ref, w_objT_ref, featsT_ref, b_col_ref,
             rel_feats_ref, w_relp_ref, b_relp_ref, boxes_ref,
             logits_out, labels_out, relbase_out,
             probT_ref, area_ref, labf_ref, selv_ref, cmax_ref, carg_ref):
    C = NUM_OBJ_CLS
    BIG = jnp.int32(2 ** 30)
    logits = jnp.dot(feats_ref[...], w_obj_ref[...],
                     preferred_element_type=jnp.float32) + b_row_ref[...]
    logits_out[...] = logits
    relbase_out[...] = jnp.dot(rel_feats_ref[...], w_relp_ref[...],
                               preferred_element_type=jnp.float32) + b_relp_ref[...]
    lT = jnp.dot(w_objT_ref[...], featsT_ref[...],
                 preferred_element_type=jnp.float32) + b_col_ref[...]
    mx = jnp.max(lT, axis=0, keepdims=True)
    e = jnp.exp(lT - mx)
    p = e / jnp.sum(e, axis=0, keepdims=True)
    cls2 = lax.broadcasted_iota(jnp.int32, (C, N_PAD), 0)
    probT_ref[...] = jnp.where(cls2 == 0, 0.0, p)

    bx = boxes_ref[...]
    area_ref[...] = ((bx[:, 2:3, :] - bx[:, 0:1, :] + 1.0) *
                     (bx[:, 3:4, :] - bx[:, 1:2, :] + 1.0))

    lane1 = lax.broadcasted_iota(jnp.int32, (1, N_PAD), 1)
    clane = lax.broadcasted_iota(jnp.int32, (1, 2 * 128), 1)
    labf_ref[...] = jnp.zeros((1, N_PAD), jnp.float32)
    selv_ref[...] = jnp.where(lane1 >= N_OBJ, 1.0, 0.0)
    cmax_ref[...] = jnp.full((1, 2 * 128), -3.0, jnp.float32)
    carg_ref[...] = jnp.zeros((1, 2 * 128), jnp.int32)

    def rescan(c):
        row = probT_ref[pl.ds(c, 1), :]
        roweff = jnp.where(selv_ref[...] > 0.0, -1.0, row)
        nm = jnp.max(roweff)
        na = jnp.min(jnp.where(roweff == nm, lane1, BIG))
        cmax_ref[...] = jnp.where(clane == c, nm, cmax_ref[...])
        carg_ref[...] = jnp.where(clane == c, na, carg_ref[...])

    def initc(c, carry):
        rescan(c)
        return carry

    lax.fori_loop(0, C, initc, 0)

    def stale_first(box):
        return jnp.min(jnp.where((carg_ref[...] == box) & (clane < C),
                                 clane, BIG))

    def body(_, carry):
        cm = cmax_ref[...]
        m = jnp.max(cm)
        flat = jnp.min(jnp.where(cm == m, carg_ref[...] * C + clane, BIG))
        box = flat // C
        cls = flat - box * C
        selm = lane1 == box
        lab = labf_ref[...]
        labf_ref[...] = jnp.where(selm & (lab == 0.0),
                                  cls.astype(jnp.float32), lab)
        selv_ref[...] = jnp.where(selm, 1.0, selv_ref[...])
        cd = boxes_ref[cls]
        ar = area_ref[cls]
        x1 = cd[0:1, :]
        y1 = cd[1:2, :]
        x2 = cd[2:3, :]
        y2 = cd[3:4, :]
        NEG = jnp.float32(-1e30)
        sx1 = jnp.max(jnp.where(selm, x1, NEG))
        sy1 = jnp.max(jnp.where(selm, y1, NEG))
        sx2 = jnp.max(jnp.where(selm, x2, NEG))
        sy2 = jnp.max(jnp.where(selm, y2, NEG))
        sar = jnp.max(jnp.where(selm, ar, NEG))
        iw = jnp.maximum(jnp.minimum(x2, sx2) - jnp.maximum(x1, sx1) + 1.0, 0.0)
        ih = jnp.maximum(jnp.minimum(y2, sy2) - jnp.maximum(y1, sy1) + 1.0, 0.0)
        inter = iw * ih
        iou = inter / (ar + sar - inter)
        iomask = iou >= NMS_THRESH
        oldrow = probT_ref[pl.ds(cls, 1), :]
        probT_ref[pl.ds(cls, 1), :] = jnp.where(iomask, 0.0, oldrow)
        carg_ref[...] = jnp.where(clane == cls, box, carg_ref[...])

        def wbody(c):
            rescan(c)
            return stale_first(box)

        lax.while_loop(lambda c: c < C, wbody, stale_first(box))
        return carry

    lax.fori_loop(0, N_OBJ, body, 0)
    labels_out[...] = labf_ref[...].astype(jnp.int32)


def _run_tc(feats, w_obj, b_row, w_objT, featsT, b_col,
            rel_feats, w_relp, b_relp, boxesT):
    return pl.pallas_call(
        _tc_body,
        out_shape=(
            jax.ShapeDtypeStruct((N_OBJ, NUM_OBJ_CLS), jnp.float32),
            jax.ShapeDtypeStruct((1, N_PAD), jnp.int32),
            jax.ShapeDtypeStruct((N_REL, REL_PAD), jnp.float32),
        ),
        scratch_shapes=[
            pltpu.VMEM((NUM_OBJ_CLS, N_PAD), jnp.float32),
            pltpu.VMEM((NUM_OBJ_CLS, 1, N_PAD), jnp.float32),
            pltpu.VMEM((1, N_PAD), jnp.float32),
            pltpu.VMEM((1, N_PAD), jnp.float32),
            pltpu.VMEM((1, 2 * 128), jnp.float32),
            pltpu.VMEM((1, 2 * 128), jnp.int32),
        ],
    )(feats, w_obj, b_row, w_objT, featsT, b_col,
      rel_feats, w_relp, b_relp, boxesT)


def _sc_body(pairs0_hbm, pairs1_hbm, labels_hbm, relbase_hbm, freq_hbm, lam_hbm,
             out_hbm, idx0_v, idx1_v, labels_v, flat_v, rows_v, rel_v, lam_v, sem):
    wid = lax.axis_index("s") * SC_NC + lax.axis_index("c")
    base = wid * BPW
    pltpu.sync_copy(pairs0_hbm.at[pl.ds(base, BPW)], idx0_v)
    pltpu.sync_copy(pairs1_hbm.at[pl.ds(base, BPW)], idx1_v)
    pltpu.sync_copy(labels_hbm, labels_v)
    pltpu.sync_copy(lam_hbm, lam_v)
    pltpu.sync_copy(relbase_hbm.at[pl.ds(base, BPW)], rel_v)

    def chunk(j, carry):
        i0 = idx0_v[pl.ds(j * SC_L, SC_L)]
        i1 = idx1_v[pl.ds(j * SC_L, SC_L)]
        s = plsc.load_gather(labels_v, [i0])
        o = plsc.load_gather(labels_v, [i1])
        flat_v[pl.ds(j * SC_L, SC_L)] = s * NUM_OBJ_CLS + o
        return carry

    lax.fori_loop(0, BPW // SC_L, chunk, 0)
    pltpu.async_copy(freq_hbm.at[flat_v], rows_v, sem).wait()
    lam = lam_v[...]

    def addrow(r, carry):
        for k in range(REL_PAD // SC_L):
            sl = pl.ds(k * SC_L, SC_L)
            rel_v[r, sl] = rel_v[r, sl] + lam * rows_v[r, sl]
        return carry

    lax.fori_loop(0, BPW, addrow, 0)
    pltpu.sync_copy(rel_v, out_hbm.at[pl.ds(base, BPW)])


def _make_sc_call():
    return pl.kernel(
        _sc_body,
        mesh=plsc.VectorSubcoreMesh(core_axis_name="c", subcore_axis_name="s"),
        compiler_params=pltpu.CompilerParams(needs_layout_passes=False),
        out_type=jax.ShapeDtypeStruct((N_REL, REL_PAD), jnp.float32),
        scratch_types=[
            pltpu.VMEM((BPW,), jnp.int32),
            pltpu.VMEM((BPW,), jnp.int32),
            pltpu.VMEM((N_PAD,), jnp.int32),
            pltpu.VMEM((BPW,), jnp.int32),
            pltpu.VMEM((BPW, REL_PAD), jnp.float32),
            pltpu.VMEM((BPW, REL_PAD), jnp.float32),
            pltpu.VMEM((SC_L,), jnp.float32),
            pltpu.SemaphoreType.DMA,
        ],
    )


def kernel(obj_feats, rel_feats, boxes_per_cls, rel_pair_idxs,
           W_obj, b_obj, W_rel, b_rel, freq_bias, freq_lambda):
    feats = obj_feats.astype(jnp.float32)
    featsT = jnp.pad(feats.T, ((0, 0), (0, N_PAD - N_OBJ)))
    w_obj = W_obj.astype(jnp.float32)
    b_row = b_obj.reshape(1, NUM_OBJ_CLS)
    b_col = b_obj.reshape(NUM_OBJ_CLS, 1)
    w_relp = jnp.pad(W_rel.astype(jnp.float32), ((0, 0), (0, REL_PAD - NUM_REL_CLS)))
    b_relp = jnp.pad(b_rel, (0, REL_PAD - NUM_REL_CLS)).reshape(1, REL_PAD)
    boxesT = jnp.pad(jnp.transpose(boxes_per_cls, (1, 2, 0)),
                     ((0, 0), (0, 0), (0, N_PAD - N_OBJ)))
    pairs0 = rel_pair_idxs[:, 0].astype(jnp.int32)
    pairs1 = rel_pair_idxs[:, 1].astype(jnp.int32)
    freq_pad = jnp.pad(freq_bias, ((0, 0), (0, REL_PAD - NUM_REL_CLS)))
    lam16 = jnp.broadcast_to(freq_lambda.astype(jnp.float32), (SC_L,))

    obj_logits, labels2d, relbase = _run_tc(
        feats, w_obj, b_row, w_obj.T, featsT, b_col,
        rel_feats.astype(jnp.float32), w_relp, b_relp, boxesT)
    labels_pad = labels2d.reshape(N_PAD)
    rel_out = _make_sc_call()(pairs0, pairs1, labels_pad, relbase,
                              freq_pad, lam16)
    return (obj_logits, rel_out[:, :NUM_REL_CLS], labels_pad[:N_OBJ])


# speedup vs baseline: 1.0709x; 1.0709x over previous
import functools

import jax
import jax.numpy as jnp
from jax import lax
from jax.experimental import pallas as pl
from jax.experimental.pallas import tpu as pltpu
from jax.experimental.pallas import tpu_sc as plsc

NUM_OBJ_CLS = 151
NUM_REL_CLS = 51
REL_PAD = 128
HIDDEN = 512
N_OBJ = 1000
N_PAD = 1024
N_REL = 4096
NMS_THRESH = 0.5

SC_NC = 2
SC_NS = 16
SC_L = 16
SC_NW = SC_NC * SC_NS
BPW = N_REL // SC_NW


def _tc_body(feats_ref, w_obj_ref, b_row_ref, w_objT_ref, featsT_ref, b_col_ref,
             rel_feats_ref, w_relp_ref, b_relp_ref, boxes_ref,
             logits_out, labels_out, relbase_out,
             probT_ref, area_ref, labf_ref, selv_ref, cmax_ref, carg_ref):
    C = NUM_OBJ_CLS
    BIG = jnp.int32(2 ** 30)
    logits = jnp.dot(feats_ref[...], w_obj_ref[...],
                     preferred_element_type=jnp.float32) + b_row_ref[...]
    logits_out[...] = logits
    relbase_out[...] = jnp.dot(rel_feats_ref[...], w_relp_ref[...],
                               preferred_element_type=jnp.float32) + b_relp_ref[...]
    lT = jnp.dot(w_objT_ref[...], featsT_ref[...],
                 preferred_element_type=jnp.float32) + b_col_ref[...]
    mx = jnp.max(lT, axis=0, keepdims=True)
    e = jnp.exp(lT - mx)
    p = e / jnp.sum(e, axis=0, keepdims=True)
    cls2 = lax.broadcasted_iota(jnp.int32, (C, N_PAD), 0)
    p = jnp.where(cls2 == 0, 0.0, p)
    probT_ref[...] = p.reshape(C, 8, 128)

    bx = boxes_ref[...]
    area_ref[...] = ((bx[:, 2, :, :] - bx[:, 0, :, :] + 1.0) *
                     (bx[:, 3, :, :] - bx[:, 1, :, :] + 1.0))

    b2 = (lax.broadcasted_iota(jnp.int32, (8, 128), 0) * 128 +
          lax.broadcasted_iota(jnp.int32, (8, 128), 1))
    clane = lax.broadcasted_iota(jnp.int32, (1, 2 * 128), 1)
    labf_ref[...] = jnp.zeros((8, 128), jnp.float32)
    selv_ref[...] = jnp.where(b2 >= N_OBJ, 1.0, 0.0)
    cmax_ref[...] = jnp.full((1, 2 * 128), -3.0, jnp.float32)
    carg_ref[...] = jnp.zeros((1, 2 * 128), jnp.int32)

    def scan_row(roweff, c):
        nm = jnp.max(roweff)
        na = jnp.min(jnp.where(roweff == nm, b2, BIG))
        cmax_ref[...] = jnp.where(clane == c, nm, cmax_ref[...])
        carg_ref[...] = jnp.where(clane == c, na, carg_ref[...])

    def rescan(c):
        row = probT_ref[c]
        scan_row(jnp.where(selv_ref[...] > 0.0, -1.0, row), c)

    def initc(c, carry):
        rescan(c)
        return carry

    lax.fori_loop(0, C, initc, 0)

    def stale_first(box):
        return jnp.min(jnp.where((carg_ref[...] == box) & (clane < C),
                                 clane, BIG))

    def body(_, carry):
        cm = cmax_ref[...]
        m = jnp.max(cm)
        flat = jnp.min(jnp.where(cm == m, carg_ref[...] * C + clane, BIG))
        box = flat // C
        cls = flat - box * C
        selm = b2 == box
        lab = labf_ref[...]
        labf_ref[...] = jnp.where(selm & (lab == 0.0),
                                  cls.astype(jnp.float32), lab)
        sv = jnp.where(selm, 1.0, selv_ref[...])
        selv_ref[...] = sv
        cd = boxes_ref[cls]
        ar = area_ref[cls]
        x1 = cd[0]
        y1 = cd[1]
        x2 = cd[2]
        y2 = cd[3]
        NEG = jnp.float32(-1e30)
        sx1 = jnp.max(jnp.where(selm, x1, NEG))
        sy1 = jnp.max(jnp.where(selm, y1, NEG))
        sx2 = jnp.max(jnp.where(selm, x2, NEG))
        sy2 = jnp.max(jnp.where(selm, y2, NEG))
        sar = jnp.max(jnp.where(selm, ar, NEG))
        iw = jnp.maximum(jnp.minimum(x2, sx2) - jnp.maximum(x1, sx1) + 1.0, 0.0)
        ih = jnp.maximum(jnp.minimum(y2, sy2) - jnp.maximum(y1, sy1) + 1.0, 0.0)
        inter = iw * ih
        iou = inter / (ar + sar - inter)
        newrow = jnp.where(iou >= NMS_THRESH, 0.0, probT_ref[cls])
        probT_ref[cls] = newrow
        scan_row(jnp.where(sv > 0.0, -1.0, newrow), cls)

        def wbody(c):
            rescan(c)
            return stale_first(box)

        lax.while_loop(lambda c: c < C, wbody, stale_first(box))
        return carry

    lax.fori_loop(0, N_OBJ, body, 0)
    labels_out[...] = labf_ref[...].astype(jnp.int32)


def _run_tc(feats, w_obj, b_row, w_objT, featsT, b_col,
            rel_feats, w_relp, b_relp, boxesT):
    return pl.pallas_call(
        _tc_body,
        out_shape=(
            jax.ShapeDtypeStruct((N_OBJ, NUM_OBJ_CLS), jnp.float32),
            jax.ShapeDtypeStruct((8, 128), jnp.int32),
            jax.ShapeDtypeStruct((N_REL, REL_PAD), jnp.float32),
        ),
        scratch_shapes=[
            pltpu.VMEM((NUM_OBJ_CLS, 8, 128), jnp.float32),
            pltpu.VMEM((NUM_OBJ_CLS, 8, 128), jnp.float32),
            pltpu.VMEM((8, 128), jnp.float32),
            pltpu.VMEM((8, 128), jnp.float32),
            pltpu.VMEM((1, 2 * 128), jnp.float32),
            pltpu.VMEM((1, 2 * 128), jnp.int32),
        ],
    )(feats, w_obj, b_row, w_objT, featsT, b_col,
      rel_feats, w_relp, b_relp, boxesT)


def _sc_body(pairs0_hbm, pairs1_hbm, labels_hbm, relbase_hbm, freq_hbm, lam_hbm,
             out_hbm, idx0_v, idx1_v, labels_v, flat_v, rows_v, rel_v, lam_v, sem):
    wid = lax.axis_index("s") * SC_NC + lax.axis_index("c")
    base = wid * BPW
    pltpu.sync_copy(pairs0_hbm.at[pl.ds(base, BPW)], idx0_v)
    pltpu.sync_copy(pairs1_hbm.at[pl.ds(base, BPW)], idx1_v)
    pltpu.sync_copy(labels_hbm, labels_v)
    pltpu.sync_copy(lam_hbm, lam_v)
    pltpu.sync_copy(relbase_hbm.at[pl.ds(base, BPW)], rel_v)

    def chunk(j, carry):
        i0 = idx0_v[pl.ds(j * SC_L, SC_L)]
        i1 = idx1_v[pl.ds(j * SC_L, SC_L)]
        s = plsc.load_gather(labels_v, [i0])
        o = plsc.load_gather(labels_v, [i1])
        flat_v[pl.ds(j * SC_L, SC_L)] = s * NUM_OBJ_CLS + o
        return carry

    lax.fori_loop(0, BPW // SC_L, chunk, 0)
    pltpu.async_copy(freq_hbm.at[flat_v], rows_v, sem).wait()
    lam = lam_v[...]

    def addrow(r, carry):
        for k in range(REL_PAD // SC_L):
            sl = pl.ds(k * SC_L, SC_L)
            rel_v[r, sl] = rel_v[r, sl] + lam * rows_v[r, sl]
        return carry

    lax.fori_loop(0, BPW, addrow, 0)
    pltpu.sync_copy(rel_v, out_hbm.at[pl.ds(base, BPW)])


def _make_sc_call():
    return pl.kernel(
        _sc_body,
        mesh=plsc.VectorSubcoreMesh(core_axis_name="c", subcore_axis_name="s"),
        compiler_params=pltpu.CompilerParams(needs_layout_passes=False),
        out_type=jax.ShapeDtypeStruct((N_REL, REL_PAD), jnp.float32),
        scratch_types=[
            pltpu.VMEM((BPW,), jnp.int32),
            pltpu.VMEM((BPW,), jnp.int32),
            pltpu.VMEM((N_PAD,), jnp.int32),
            pltpu.VMEM((BPW,), jnp.int32),
            pltpu.VMEM((BPW, REL_PAD), jnp.float32),
            pltpu.VMEM((BPW, REL_PAD), jnp.float32),
            pltpu.VMEM((SC_L,), jnp.float32),
            pltpu.SemaphoreType.DMA,
        ],
    )


def kernel(obj_feats, rel_feats, boxes_per_cls, rel_pair_idxs,
           W_obj, b_obj, W_rel, b_rel, freq_bias, freq_lambda):
    feats = obj_feats.astype(jnp.float32)
    featsT = jnp.pad(feats.T, ((0, 0), (0, N_PAD - N_OBJ)))
    w_obj = W_obj.astype(jnp.float32)
    b_row = b_obj.reshape(1, NUM_OBJ_CLS)
    b_col = b_obj.reshape(NUM_OBJ_CLS, 1)
    w_relp = jnp.pad(W_rel.astype(jnp.float32), ((0, 0), (0, REL_PAD - NUM_REL_CLS)))
    b_relp = jnp.pad(b_rel, (0, REL_PAD - NUM_REL_CLS)).reshape(1, REL_PAD)
    boxesT = jnp.pad(jnp.transpose(boxes_per_cls, (1, 2, 0)),
                     ((0, 0), (0, 0), (0, N_PAD - N_OBJ))
                     ).reshape(NUM_OBJ_CLS, 4, 8, 128)
    pairs0 = rel_pair_idxs[:, 0].astype(jnp.int32)
    pairs1 = rel_pair_idxs[:, 1].astype(jnp.int32)
    freq_pad = jnp.pad(freq_bias, ((0, 0), (0, REL_PAD - NUM_REL_CLS)))
    lam16 = jnp.broadcast_to(freq_lambda.astype(jnp.float32), (SC_L,))

    obj_logits, labels2d, relbase = _run_tc(
        feats, w_obj, b_row, w_obj.T, featsT, b_col,
        rel_feats.astype(jnp.float32), w_relp, b_relp, boxesT)
    labels_pad = labels2d.reshape(N_PAD)
    rel_out = _make_sc_call()(pairs0, pairs1, labels_pad, relbase,
                              freq_pad, lam16)
    return (obj_logits, rel_out[:, :NUM_REL_CLS], labels_pad[:N_OBJ])


# speedup vs baseline: 1.5812x; 1.4765x over previous
import functools

import jax
import jax.numpy as jnp
from jax import lax
from jax.experimental import pallas as pl
from jax.experimental.pallas import tpu as pltpu
from jax.experimental.pallas import tpu_sc as plsc

NUM_OBJ_CLS = 151
NUM_REL_CLS = 51
REL_PAD = 128
HIDDEN = 512
N_OBJ = 1000
N_PAD = 1024
N_REL = 4096
NMS_THRESH = 0.5

SC_NC = 2
SC_NS = 16
SC_L = 16
SC_NW = SC_NC * SC_NS
BPW = N_REL // SC_NW


def _tc_body(feats_ref, w_obj_ref, b_row_ref, w_objT_ref, featsT_ref, b_col_ref,
             rel_feats_ref, w_relp_ref, b_relp_ref, boxes_ref,
             logits_out, labels_out, relbase_out,
             probT_ref, area_ref, labf_ref, selv_ref, cmax_ref, carg_ref):
    C = NUM_OBJ_CLS
    BIG = jnp.int32(2 ** 30)
    logits = jnp.dot(feats_ref[...], w_obj_ref[...],
                     preferred_element_type=jnp.float32) + b_row_ref[...]
    logits_out[...] = logits
    relbase_out[...] = jnp.dot(rel_feats_ref[...], w_relp_ref[...],
                               preferred_element_type=jnp.float32) + b_relp_ref[...]
    lT = jnp.dot(w_objT_ref[...], featsT_ref[...],
                 preferred_element_type=jnp.float32) + b_col_ref[...]
    mx = jnp.max(lT, axis=0, keepdims=True)
    e = jnp.exp(lT - mx)
    p = e / jnp.sum(e, axis=0, keepdims=True)
    cls2 = lax.broadcasted_iota(jnp.int32, (C, N_PAD), 0)
    p = jnp.where(cls2 == 0, 0.0, p)
    probT_ref[...] = p.reshape(C, 8, 128)

    bx = boxes_ref[...]
    area_ref[...] = ((bx[:, 2, :, :] - bx[:, 0, :, :] + 1.0) *
                     (bx[:, 3, :, :] - bx[:, 1, :, :] + 1.0))

    b2 = (lax.broadcasted_iota(jnp.int32, (8, 128), 0) * 128 +
          lax.broadcasted_iota(jnp.int32, (8, 128), 1))
    b2f = b2.astype(jnp.float32)
    clane = lax.broadcasted_iota(jnp.int32, (1, 2 * 128), 1)
    clanef = clane.astype(jnp.float32)
    BIGF = jnp.float32(3e7)
    Cf = jnp.float32(C)
    labf_ref[...] = jnp.zeros((8, 128), jnp.float32)
    selv_ref[...] = jnp.where(b2 >= N_OBJ, 1.0, 0.0)
    cmax_ref[...] = jnp.full((1, 2 * 128), -3.0, jnp.float32)
    carg_ref[...] = jnp.zeros((1, 2 * 128), jnp.float32)

    def scan_row(roweff, c):
        nm = jnp.max(roweff)
        na = jnp.min(jnp.where(roweff == nm, b2f, BIGF))
        cmask = clane == c
        cmax_ref[...] = jnp.where(cmask, nm, cmax_ref[...])
        carg_ref[...] = jnp.where(cmask, na, carg_ref[...])

    def rescan(c):
        row = probT_ref[c]
        scan_row(jnp.where(selv_ref[...] > 0.0, -1.0, row), c)

    def initc(c, carry):
        rescan(c)
        return carry

    lax.fori_loop(0, C, initc, 0)

    def body(_, carry):
        cm = cmax_ref[...]
        cg = carg_ref[...]
        m = jnp.max(cm)
        flatf = jnp.min(jnp.where(cm == m, cg * Cf + clanef, BIGF))
        flat = flatf.astype(jnp.int32)
        box = flat // C
        cls = flat - box * C
        boxf = box.astype(jnp.float32)
        stale0f = jnp.min(jnp.where((cg == boxf) & (clanef < Cf)
                                    & (clane != cls), clanef, BIGF))
        selm = b2 == box
        lab = labf_ref[...]
        labf_ref[...] = jnp.where(selm & (lab == 0.0),
                                  cls.astype(jnp.float32), lab)
        sv = jnp.where(selm, 1.0, selv_ref[...])
        selv_ref[...] = sv
        cd = boxes_ref[cls]
        ar = area_ref[cls]
        x1 = cd[0]
        y1 = cd[1]
        x2 = cd[2]
        y2 = cd[3]
        NEG = jnp.float32(-1e30)
        sx1 = jnp.max(jnp.where(selm, x1, NEG))
        sy1 = jnp.max(jnp.where(selm, y1, NEG))
        sx2 = jnp.max(jnp.where(selm, x2, NEG))
        sy2 = jnp.max(jnp.where(selm, y2, NEG))
        sar = jnp.max(jnp.where(selm, ar, NEG))
        iw = jnp.maximum(jnp.minimum(x2, sx2) - jnp.maximum(x1, sx1) + 1.0, 0.0)
        ih = jnp.maximum(jnp.minimum(y2, sy2) - jnp.maximum(y1, sy1) + 1.0, 0.0)
        inter = iw * ih
        iou = inter / (ar + sar - inter)
        newrow = jnp.where(iou >= NMS_THRESH, 0.0, probT_ref[cls])
        probT_ref[cls] = newrow
        scan_row(jnp.where(sv > 0.0, -1.0, newrow), cls)

        def wbody(c):
            rescan(c)
            cgl = carg_ref[...]
            return jnp.min(jnp.where((cgl == boxf) & (clanef < Cf),
                                     clanef, BIGF)).astype(jnp.int32)

        lax.while_loop(lambda c: c < C, wbody, stale0f.astype(jnp.int32))
        return carry

    lax.fori_loop(0, N_OBJ, body, 0)
    labels_out[...] = labf_ref[...].astype(jnp.int32)


def _run_tc(feats, w_obj, b_row, w_objT, featsT, b_col,
            rel_feats, w_relp, b_relp, boxesT):
    return pl.pallas_call(
        _tc_body,
        out_shape=(
            jax.ShapeDtypeStruct((N_OBJ, NUM_OBJ_CLS), jnp.float32),
            jax.ShapeDtypeStruct((8, 128), jnp.int32),
            jax.ShapeDtypeStruct((N_REL, REL_PAD), jnp.float32),
        ),
        scratch_shapes=[
            pltpu.VMEM((NUM_OBJ_CLS, 8, 128), jnp.float32),
            pltpu.VMEM((NUM_OBJ_CLS, 8, 128), jnp.float32),
            pltpu.VMEM((8, 128), jnp.float32),
            pltpu.VMEM((8, 128), jnp.float32),
            pltpu.VMEM((1, 2 * 128), jnp.float32),
            pltpu.VMEM((1, 2 * 128), jnp.float32),
        ],
    )(feats, w_obj, b_row, w_objT, featsT, b_col,
      rel_feats, w_relp, b_relp, boxesT)


def _sc_body(pairs0_hbm, pairs1_hbm, labels_hbm, relbase_hbm, freq_hbm, lam_hbm,
             out_hbm, idx0_v, idx1_v, labels_v, flat_v, rows_v, rel_v, lam_v, sem):
    wid = lax.axis_index("s") * SC_NC + lax.axis_index("c")
    base = wid * BPW
    pltpu.sync_copy(pairs0_hbm.at[pl.ds(base, BPW)], idx0_v)
    pltpu.sync_copy(pairs1_hbm.at[pl.ds(base, BPW)], idx1_v)
    pltpu.sync_copy(labels_hbm, labels_v)
    pltpu.sync_copy(lam_hbm, lam_v)
    pltpu.sync_copy(relbase_hbm.at[pl.ds(base, BPW)], rel_v)

    def chunk(j, carry):
        i0 = idx0_v[pl.ds(j * SC_L, SC_L)]
        i1 = idx1_v[pl.ds(j * SC_L, SC_L)]
        s = plsc.load_gather(labels_v, [i0])
        o = plsc.load_gather(labels_v, [i1])
        flat_v[pl.ds(j * SC_L, SC_L)] = s * NUM_OBJ_CLS + o
        return carry

    lax.fori_loop(0, BPW // SC_L, chunk, 0)
    pltpu.async_copy(freq_hbm.at[flat_v], rows_v, sem).wait()
    lam = lam_v[...]

    def addrow(r, carry):
        for k in range(REL_PAD // SC_L):
            sl = pl.ds(k * SC_L, SC_L)
            rel_v[r, sl] = rel_v[r, sl] + lam * rows_v[r, sl]
        return carry

    lax.fori_loop(0, BPW, addrow, 0)
    pltpu.sync_copy(rel_v, out_hbm.at[pl.ds(base, BPW)])


def _make_sc_call():
    return pl.kernel(
        _sc_body,
        mesh=plsc.VectorSubcoreMesh(core_axis_name="c", subcore_axis_name="s"),
        compiler_params=pltpu.CompilerParams(needs_layout_passes=False),
        out_type=jax.ShapeDtypeStruct((N_REL, REL_PAD), jnp.float32),
        scratch_types=[
            pltpu.VMEM((BPW,), jnp.int32),
            pltpu.VMEM((BPW,), jnp.int32),
            pltpu.VMEM((N_PAD,), jnp.int32),
            pltpu.VMEM((BPW,), jnp.int32),
            pltpu.VMEM((BPW, REL_PAD), jnp.float32),
            pltpu.VMEM((BPW, REL_PAD), jnp.float32),
            pltpu.VMEM((SC_L,), jnp.float32),
            pltpu.SemaphoreType.DMA,
        ],
    )


def kernel(obj_feats, rel_feats, boxes_per_cls, rel_pair_idxs,
           W_obj, b_obj, W_rel, b_rel, freq_bias, freq_lambda):
    feats = obj_feats.astype(jnp.float32)
    featsT = jnp.pad(feats.T, ((0, 0), (0, N_PAD - N_OBJ)))
    w_obj = W_obj.astype(jnp.float32)
    b_row = b_obj.reshape(1, NUM_OBJ_CLS)
    b_col = b_obj.reshape(NUM_OBJ_CLS, 1)
    w_relp = jnp.pad(W_rel.astype(jnp.float32), ((0, 0), (0, REL_PAD - NUM_REL_CLS)))
    b_relp = jnp.pad(b_rel, (0, REL_PAD - NUM_REL_CLS)).reshape(1, REL_PAD)
    boxesT = jnp.pad(jnp.transpose(boxes_per_cls, (1, 2, 0)),
                     ((0, 0), (0, 0), (0, N_PAD - N_OBJ))
                     ).reshape(NUM_OBJ_CLS, 4, 8, 128)
    pairs0 = rel_pair_idxs[:, 0].astype(jnp.int32)
    pairs1 = rel_pair_idxs[:, 1].astype(jnp.int32)
    freq_pad = jnp.pad(freq_bias, ((0, 0), (0, REL_PAD - NUM_REL_CLS)))
    lam16 = jnp.broadcast_to(freq_lambda.astype(jnp.float32), (SC_L,))

    obj_logits, labels2d, relbase = _run_tc(
        feats, w_obj, b_row, w_obj.T, featsT, b_col,
        rel_feats.astype(jnp.float32), w_relp, b_relp, boxesT)
    labels_pad = labels2d.reshape(N_PAD)
    rel_out = _make_sc_call()(pairs0, pairs1, labels_pad, relbase,
                              freq_pad, lam16)
    return (obj_logits, rel_out[:, :NUM_REL_CLS], labels_pad[:N_OBJ])


# speedup vs baseline: 3.3131x; 2.0954x over previous
import functools

import jax
import jax.numpy as jnp
from jax import lax
from jax.experimental import pallas as pl
from jax.experimental.pallas import tpu as pltpu
from jax.experimental.pallas import tpu_sc as plsc

NUM_OBJ_CLS = 151
NUM_REL_CLS = 51
REL_PAD = 128
HIDDEN = 512
N_OBJ = 1000
N_PAD = 1024
N_REL = 4096
NMS_THRESH = 0.5

SC_NC = 2
SC_NS = 16
SC_L = 16
SC_NW = SC_NC * SC_NS
BPW = N_REL // SC_NW


def _tc_body(feats_ref, w_obj_ref, b_row_ref, w_objT_ref, featsT_ref, b_col_ref,
             rel_feats_ref, w_relp_ref, b_relp_ref, boxes_ref,
             logits_out, labels_out, relbase_out,
             probT_ref, area_ref, labf_ref, selv_ref, cmax_ref, carg_ref):
    C = NUM_OBJ_CLS
    BIG = jnp.int32(2 ** 30)
    logits = jnp.dot(feats_ref[...], w_obj_ref[...],
                     preferred_element_type=jnp.float32) + b_row_ref[...]
    logits_out[...] = logits
    relbase_out[...] = jnp.dot(rel_feats_ref[...], w_relp_ref[...],
                               preferred_element_type=jnp.float32) + b_relp_ref[...]
    lT = jnp.dot(w_objT_ref[...], featsT_ref[...],
                 preferred_element_type=jnp.float32) + b_col_ref[...]
    mx = jnp.max(lT, axis=0, keepdims=True)
    e = jnp.exp(lT - mx)
    p = e / jnp.sum(e, axis=0, keepdims=True)
    cls2 = lax.broadcasted_iota(jnp.int32, (C, N_PAD), 0)
    p = jnp.where(cls2 == 0, 0.0, p)
    probT_ref[...] = p.reshape(C, 8, 128)

    bx = boxes_ref[...]
    area_ref[...] = ((bx[:, 2, :, :] - bx[:, 0, :, :] + 1.0) *
                     (bx[:, 3, :, :] - bx[:, 1, :, :] + 1.0))

    b2 = (lax.broadcasted_iota(jnp.int32, (8, 128), 0) * 128 +
          lax.broadcasted_iota(jnp.int32, (8, 128), 1))
    b2f = b2.astype(jnp.float32)
    BIGF = jnp.float32(3e7)
    Cf = jnp.float32(C)
    labf_ref[...] = jnp.zeros((8, 128), jnp.float32)
    selv_ref[...] = jnp.where(b2 >= N_OBJ, 1.0, 0.0)

    NCH = 8

    def body(_, carry):
        svm = selv_ref[...] > 0.0
        accv = [jnp.full((8, 128), -4.0, jnp.float32) for _ in range(NCH)]
        acci = [jnp.zeros((8, 128), jnp.float32) for _ in range(NCH)]
        for c in range(C):
            row = jnp.where(svm, -1.0, probT_ref[c])
            k = c % NCH
            gt = row > accv[k]
            acci[k] = jnp.where(gt, jnp.float32(c), acci[k])
            accv[k] = jnp.maximum(row, accv[k])
        while len(accv) > 1:
            nv, ni = [], []
            for j in range(0, len(accv), 2):
                va, ia, vb, ib = accv[j], acci[j], accv[j + 1], acci[j + 1]
                gt = va > vb
                eq = va == vb
                ni.append(jnp.where(gt, ia,
                                    jnp.where(eq, jnp.minimum(ia, ib), ib)))
                nv.append(jnp.maximum(va, vb))
            accv, acci = nv, ni
        av, ai = accv[0], acci[0]
        m = jnp.max(av)
        flatf = jnp.min(jnp.where(av == m, b2f * Cf + ai, BIGF))
        flat = flatf.astype(jnp.int32)
        box = flat // C
        cls = flat - box * C
        selm = b2 == box
        lab = labf_ref[...]
        labf_ref[...] = jnp.where(selm & (lab == 0.0),
                                  cls.astype(jnp.float32), lab)
        selv_ref[...] = jnp.where(selm, 1.0, selv_ref[...])
        cd = boxes_ref[cls]
        ar = area_ref[cls]
        x1 = cd[0]
        y1 = cd[1]
        x2 = cd[2]
        y2 = cd[3]
        NEG = jnp.float32(-1e30)
        sx1 = jnp.max(jnp.where(selm, x1, NEG))
        sy1 = jnp.max(jnp.where(selm, y1, NEG))
        sx2 = jnp.max(jnp.where(selm, x2, NEG))
        sy2 = jnp.max(jnp.where(selm, y2, NEG))
        sar = jnp.max(jnp.where(selm, ar, NEG))
        iw = jnp.maximum(jnp.minimum(x2, sx2) - jnp.maximum(x1, sx1) + 1.0, 0.0)
        ih = jnp.maximum(jnp.minimum(y2, sy2) - jnp.maximum(y1, sy1) + 1.0, 0.0)
        inter = iw * ih
        iou = inter / (ar + sar - inter)
        newrow = jnp.where(iou >= NMS_THRESH, 0.0, probT_ref[cls])
        probT_ref[cls] = newrow
        return carry

    lax.fori_loop(0, N_OBJ, body, 0)
    labels_out[...] = labf_ref[...].astype(jnp.int32)


def _run_tc(feats, w_obj, b_row, w_objT, featsT, b_col,
            rel_feats, w_relp, b_relp, boxesT):
    return pl.pallas_call(
        _tc_body,
        out_shape=(
            jax.ShapeDtypeStruct((N_OBJ, NUM_OBJ_CLS), jnp.float32),
            jax.ShapeDtypeStruct((8, 128), jnp.int32),
            jax.ShapeDtypeStruct((N_REL, REL_PAD), jnp.float32),
        ),
        scratch_shapes=[
            pltpu.VMEM((NUM_OBJ_CLS, 8, 128), jnp.float32),
            pltpu.VMEM((NUM_OBJ_CLS, 8, 128), jnp.float32),
            pltpu.VMEM((8, 128), jnp.float32),
            pltpu.VMEM((8, 128), jnp.float32),
            pltpu.VMEM((1, 2 * 128), jnp.float32),
            pltpu.VMEM((1, 2 * 128), jnp.float32),
        ],
    )(feats, w_obj, b_row, w_objT, featsT, b_col,
      rel_feats, w_relp, b_relp, boxesT)


def _sc_body(pairs0_hbm, pairs1_hbm, labels_hbm, relbase_hbm, freq_hbm, lam_hbm,
             out_hbm, idx0_v, idx1_v, labels_v, flat_v, rows_v, rel_v, lam_v, sem):
    wid = lax.axis_index("s") * SC_NC + lax.axis_index("c")
    base = wid * BPW
    pltpu.sync_copy(pairs0_hbm.at[pl.ds(base, BPW)], idx0_v)
    pltpu.sync_copy(pairs1_hbm.at[pl.ds(base, BPW)], idx1_v)
    pltpu.sync_copy(labels_hbm, labels_v)
    pltpu.sync_copy(lam_hbm, lam_v)
    pltpu.sync_copy(relbase_hbm.at[pl.ds(base, BPW)], rel_v)

    def chunk(j, carry):
        i0 = idx0_v[pl.ds(j * SC_L, SC_L)]
        i1 = idx1_v[pl.ds(j * SC_L, SC_L)]
        s = plsc.load_gather(labels_v, [i0])
        o = plsc.load_gather(labels_v, [i1])
        flat_v[pl.ds(j * SC_L, SC_L)] = s * NUM_OBJ_CLS + o
        return carry

    lax.fori_loop(0, BPW // SC_L, chunk, 0)
    pltpu.async_copy(freq_hbm.at[flat_v], rows_v, sem).wait()
    lam = lam_v[...]

    def addrow(r, carry):
        for k in range(REL_PAD // SC_L):
            sl = pl.ds(k * SC_L, SC_L)
            rel_v[r, sl] = rel_v[r, sl] + lam * rows_v[r, sl]
        return carry

    lax.fori_loop(0, BPW, addrow, 0)
    pltpu.sync_copy(rel_v, out_hbm.at[pl.ds(base, BPW)])


def _make_sc_call():
    return pl.kernel(
        _sc_body,
        mesh=plsc.VectorSubcoreMesh(core_axis_name="c", subcore_axis_name="s"),
        compiler_params=pltpu.CompilerParams(needs_layout_passes=False),
        out_type=jax.ShapeDtypeStruct((N_REL, REL_PAD), jnp.float32),
        scratch_types=[
            pltpu.VMEM((BPW,), jnp.int32),
            pltpu.VMEM((BPW,), jnp.int32),
            pltpu.VMEM((N_PAD,), jnp.int32),
            pltpu.VMEM((BPW,), jnp.int32),
            pltpu.VMEM((BPW, REL_PAD), jnp.float32),
            pltpu.VMEM((BPW, REL_PAD), jnp.float32),
            pltpu.VMEM((SC_L,), jnp.float32),
            pltpu.SemaphoreType.DMA,
        ],
    )


def kernel(obj_feats, rel_feats, boxes_per_cls, rel_pair_idxs,
           W_obj, b_obj, W_rel, b_rel, freq_bias, freq_lambda):
    feats = obj_feats.astype(jnp.float32)
    featsT = jnp.pad(feats.T, ((0, 0), (0, N_PAD - N_OBJ)))
    w_obj = W_obj.astype(jnp.float32)
    b_row = b_obj.reshape(1, NUM_OBJ_CLS)
    b_col = b_obj.reshape(NUM_OBJ_CLS, 1)
    w_relp = jnp.pad(W_rel.astype(jnp.float32), ((0, 0), (0, REL_PAD - NUM_REL_CLS)))
    b_relp = jnp.pad(b_rel, (0, REL_PAD - NUM_REL_CLS)).reshape(1, REL_PAD)
    boxesT = jnp.pad(jnp.transpose(boxes_per_cls, (1, 2, 0)),
                     ((0, 0), (0, 0), (0, N_PAD - N_OBJ))
                     ).reshape(NUM_OBJ_CLS, 4, 8, 128)
    pairs0 = rel_pair_idxs[:, 0].astype(jnp.int32)
    pairs1 = rel_pair_idxs[:, 1].astype(jnp.int32)
    freq_pad = jnp.pad(freq_bias, ((0, 0), (0, REL_PAD - NUM_REL_CLS)))
    lam16 = jnp.broadcast_to(freq_lambda.astype(jnp.float32), (SC_L,))

    obj_logits, labels2d, relbase = _run_tc(
        feats, w_obj, b_row, w_obj.T, featsT, b_col,
        rel_feats.astype(jnp.float32), w_relp, b_relp, boxesT)
    labels_pad = labels2d.reshape(N_PAD)
    rel_out = _make_sc_call()(pairs0, pairs1, labels_pad, relbase,
                              freq_pad, lam16)
    return (obj_logits, rel_out[:, :NUM_REL_CLS], labels_pad[:N_OBJ])


# speedup vs baseline: 4.1416x; 1.2501x over previous
import functools

import jax
import jax.numpy as jnp
from jax import lax
from jax.experimental import pallas as pl
from jax.experimental.pallas import tpu as pltpu
from jax.experimental.pallas import tpu_sc as plsc

NUM_OBJ_CLS = 151
NUM_REL_CLS = 51
REL_PAD = 128
HIDDEN = 512
N_OBJ = 1000
N_PAD = 1024
N_REL = 4096
NMS_THRESH = 0.5

SC_NC = 2
SC_NS = 16
SC_L = 16
SC_NW = SC_NC * SC_NS
BPW = N_REL // SC_NW


def _tc_body(feats_ref, w_obj_ref, b_row_ref, w_objT_ref, featsT_ref, b_col_ref,
             rel_feats_ref, w_relp_ref, b_relp_ref, boxes_ref,
             logits_out, labels_out, relbase_out,
             probT_ref, area_ref, labf_ref, selv_ref, cmax_ref, carg_ref):
    C = NUM_OBJ_CLS
    BIG = jnp.int32(2 ** 30)
    logits = jnp.dot(feats_ref[...], w_obj_ref[...],
                     preferred_element_type=jnp.float32) + b_row_ref[...]
    logits_out[...] = logits
    relbase_out[...] = jnp.dot(rel_feats_ref[...], w_relp_ref[...],
                               preferred_element_type=jnp.float32) + b_relp_ref[...]
    lT = jnp.dot(w_objT_ref[...], featsT_ref[...],
                 preferred_element_type=jnp.float32) + b_col_ref[...]
    mx = jnp.max(lT, axis=0, keepdims=True)
    e = jnp.exp(lT - mx)
    p = e / jnp.sum(e, axis=0, keepdims=True)
    cls2 = lax.broadcasted_iota(jnp.int32, (C, N_PAD), 0)
    p = jnp.where(cls2 == 0, 0.0, p)
    probT_ref[...] = p.reshape(C, 8, 128)

    bx = boxes_ref[...]
    area_ref[...] = ((bx[:, 2, :, :] - bx[:, 0, :, :] + 1.0) *
                     (bx[:, 3, :, :] - bx[:, 1, :, :] + 1.0))

    b2 = (lax.broadcasted_iota(jnp.int32, (8, 128), 0) * 128 +
          lax.broadcasted_iota(jnp.int32, (8, 128), 1))
    b2f = b2.astype(jnp.float32)
    BIGF = jnp.float32(3e7)
    Cf = jnp.float32(C)
    labf_ref[...] = jnp.zeros((8, 128), jnp.float32)
    selv_ref[...] = jnp.where(b2 >= N_OBJ, 1.0, 0.0)

    NCH = 8

    def pick_updates(box, cls, selm, lab_ok):
        lab = labf_ref[...]
        labf_ref[...] = jnp.where(lab_ok & selm & (lab == 0.0),
                                  cls.astype(jnp.float32), lab)
        selv_ref[...] = jnp.where(lab_ok & selm, 1.0, selv_ref[...])
        cd = boxes_ref[cls]
        ar = area_ref[cls]
        x1 = cd[0]
        y1 = cd[1]
        x2 = cd[2]
        y2 = cd[3]
        NEG = jnp.float32(-1e30)
        sx1 = jnp.max(jnp.where(selm, x1, NEG))
        sy1 = jnp.max(jnp.where(selm, y1, NEG))
        sx2 = jnp.max(jnp.where(selm, x2, NEG))
        sy2 = jnp.max(jnp.where(selm, y2, NEG))
        sar = jnp.max(jnp.where(selm, ar, NEG))
        iw = jnp.maximum(jnp.minimum(x2, sx2) - jnp.maximum(x1, sx1) + 1.0, 0.0)
        ih = jnp.maximum(jnp.minimum(y2, sy2) - jnp.maximum(y1, sy1) + 1.0, 0.0)
        inter = iw * ih
        iou = inter / (ar + sar - inter)
        iomask = iou >= NMS_THRESH
        newrow = jnp.where(lab_ok & iomask, 0.0, probT_ref[cls])
        probT_ref[cls] = newrow
        return iou

    def body(picks):
        svm = selv_ref[...] > 0.0
        accv = [jnp.full((8, 128), -4.0, jnp.float32) for _ in range(NCH)]
        acci = [jnp.zeros((8, 128), jnp.float32) for _ in range(NCH)]
        for c in range(C):
            row = jnp.where(svm, -1.0, probT_ref[c])
            k = c % NCH
            gt = row > accv[k]
            acci[k] = jnp.where(gt, jnp.float32(c), acci[k])
            accv[k] = jnp.maximum(row, accv[k])
        while len(accv) > 1:
            nv, ni = [], []
            for j in range(0, len(accv), 2):
                va, ia, vb, ib = accv[j], acci[j], accv[j + 1], acci[j + 1]
                gt = va > vb
                eq = va == vb
                ni.append(jnp.where(gt, ia,
                                    jnp.where(eq, jnp.minimum(ia, ib), ib)))
                nv.append(jnp.maximum(va, vb))
            accv, acci = nv, ni
        av, ai = accv[0], acci[0]
        keyf = b2f * Cf + ai
        m = jnp.max(av)
        flat = jnp.min(jnp.where(av == m, keyf, BIGF)).astype(jnp.int32)
        box = flat // C
        cls = flat - box * C
        selm = b2 == box
        iou1 = pick_updates(box, cls, selm, True)
        av2 = jnp.where(selm, -4.0, av)
        m2 = jnp.max(av2)
        flat2 = jnp.min(jnp.where(av2 == m2, keyf, BIGF)).astype(jnp.int32)
        box2 = flat2 // C
        cls2 = flat2 - box2 * C
        selm2 = b2 == box2
        iou1_at2 = jnp.max(jnp.where(selm2, iou1, 0.0))
        ok2 = ((cls2 != cls) | (iou1_at2 < NMS_THRESH)) & (picks < N_OBJ - 2)
        pick_updates(box2, cls2, selm2, ok2)
        return picks + 1 + ok2.astype(jnp.int32)

    lax.while_loop(lambda picks: picks < N_OBJ, body, jnp.int32(0))
    labels_out[...] = labf_ref[...].astype(jnp.int32)


def _run_tc(feats, w_obj, b_row, w_objT, featsT, b_col,
            rel_feats, w_relp, b_relp, boxesT):
    return pl.pallas_call(
        _tc_body,
        out_shape=(
            jax.ShapeDtypeStruct((N_OBJ, NUM_OBJ_CLS), jnp.float32),
            jax.ShapeDtypeStruct((8, 128), jnp.int32),
            jax.ShapeDtypeStruct((N_REL, REL_PAD), jnp.float32),
        ),
        scratch_shapes=[
            pltpu.VMEM((NUM_OBJ_CLS, 8, 128), jnp.float32),
            pltpu.VMEM((NUM_OBJ_CLS, 8, 128), jnp.float32),
            pltpu.VMEM((8, 128), jnp.float32),
            pltpu.VMEM((8, 128), jnp.float32),
            pltpu.VMEM((1, 2 * 128), jnp.float32),
            pltpu.VMEM((1, 2 * 128), jnp.float32),
        ],
    )(feats, w_obj, b_row, w_objT, featsT, b_col,
      rel_feats, w_relp, b_relp, boxesT)


def _sc_body(pairs0_hbm, pairs1_hbm, labels_hbm, relbase_hbm, freq_hbm, lam_hbm,
             out_hbm, idx0_v, idx1_v, labels_v, flat_v, rows_v, rel_v, lam_v, sem):
    wid = lax.axis_index("s") * SC_NC + lax.axis_index("c")
    base = wid * BPW
    pltpu.sync_copy(pairs0_hbm.at[pl.ds(base, BPW)], idx0_v)
    pltpu.sync_copy(pairs1_hbm.at[pl.ds(base, BPW)], idx1_v)
    pltpu.sync_copy(labels_hbm, labels_v)
    pltpu.sync_copy(lam_hbm, lam_v)
    pltpu.sync_copy(relbase_hbm.at[pl.ds(base, BPW)], rel_v)

    def chunk(j, carry):
        i0 = idx0_v[pl.ds(j * SC_L, SC_L)]
        i1 = idx1_v[pl.ds(j * SC_L, SC_L)]
        s = plsc.load_gather(labels_v, [i0])
        o = plsc.load_gather(labels_v, [i1])
        flat_v[pl.ds(j * SC_L, SC_L)] = s * NUM_OBJ_CLS + o
        return carry

    lax.fori_loop(0, BPW // SC_L, chunk, 0)
    pltpu.async_copy(freq_hbm.at[flat_v], rows_v, sem).wait()
    lam = lam_v[...]

    def addrow(r, carry):
        for k in range(REL_PAD // SC_L):
            sl = pl.ds(k * SC_L, SC_L)
            rel_v[r, sl] = rel_v[r, sl] + lam * rows_v[r, sl]
        return carry

    lax.fori_loop(0, BPW, addrow, 0)
    pltpu.sync_copy(rel_v, out_hbm.at[pl.ds(base, BPW)])


def _make_sc_call():
    return pl.kernel(
        _sc_body,
        mesh=plsc.VectorSubcoreMesh(core_axis_name="c", subcore_axis_name="s"),
        compiler_params=pltpu.CompilerParams(needs_layout_passes=False),
        out_type=jax.ShapeDtypeStruct((N_REL, REL_PAD), jnp.float32),
        scratch_types=[
            pltpu.VMEM((BPW,), jnp.int32),
            pltpu.VMEM((BPW,), jnp.int32),
            pltpu.VMEM((N_PAD,), jnp.int32),
            pltpu.VMEM((BPW,), jnp.int32),
            pltpu.VMEM((BPW, REL_PAD), jnp.float32),
            pltpu.VMEM((BPW, REL_PAD), jnp.float32),
            pltpu.VMEM((SC_L,), jnp.float32),
            pltpu.SemaphoreType.DMA,
        ],
    )


def kernel(obj_feats, rel_feats, boxes_per_cls, rel_pair_idxs,
           W_obj, b_obj, W_rel, b_rel, freq_bias, freq_lambda):
    feats = obj_feats.astype(jnp.float32)
    featsT = jnp.pad(feats.T, ((0, 0), (0, N_PAD - N_OBJ)))
    w_obj = W_obj.astype(jnp.float32)
    b_row = b_obj.reshape(1, NUM_OBJ_CLS)
    b_col = b_obj.reshape(NUM_OBJ_CLS, 1)
    w_relp = jnp.pad(W_rel.astype(jnp.float32), ((0, 0), (0, REL_PAD - NUM_REL_CLS)))
    b_relp = jnp.pad(b_rel, (0, REL_PAD - NUM_REL_CLS)).reshape(1, REL_PAD)
    boxesT = jnp.pad(jnp.transpose(boxes_per_cls, (1, 2, 0)),
                     ((0, 0), (0, 0), (0, N_PAD - N_OBJ))
                     ).reshape(NUM_OBJ_CLS, 4, 8, 128)
    pairs0 = rel_pair_idxs[:, 0].astype(jnp.int32)
    pairs1 = rel_pair_idxs[:, 1].astype(jnp.int32)
    freq_pad = jnp.pad(freq_bias, ((0, 0), (0, REL_PAD - NUM_REL_CLS)))
    lam16 = jnp.broadcast_to(freq_lambda.astype(jnp.float32), (SC_L,))

    obj_logits, labels2d, relbase = _run_tc(
        feats, w_obj, b_row, w_obj.T, featsT, b_col,
        rel_feats.astype(jnp.float32), w_relp, b_relp, boxesT)
    labels_pad = labels2d.reshape(N_PAD)
    rel_out = _make_sc_call()(pairs0, pairs1, labels_pad, relbase,
                              freq_pad, lam16)
    return (obj_logits, rel_out[:, :NUM_REL_CLS], labels_pad[:N_OBJ])
